# baseline (speedup 1.0000x reference)
import functools

import jax
import jax.numpy as jnp
from jax import lax
from jax.experimental import pallas as pl
from jax.experimental.pallas import tpu as pltpu
from jax.experimental.pallas import tpu_sc as plsc

N_NODES = 10000
FEAT = 128
OE = 256
EDGE_BLOCK = 12800
NODE_BLOCK = 2000
CHUNK = 256
KSUB = CHUNK // 128
N_TILES = 32
N_PAD = 10240
HALF = N_PAD // 2
ACC_ROWS = HALF + 8
ZSUB = HALF // 16
MAX_TILE_CHUNKS = 80
IDX_ROWS_CORE = 2816


def _edge_body(rbft_ref, x_ref, w_ref, h_ref):
    s = lax.dot_general(rbft_ref[...], w_ref[...],
                        dimension_numbers=(((0,), (0,)), ((), ())),
                        preferred_element_type=jnp.float32)
    h_ref[...] = s * x_ref[...]


def _mlp_body(p_ref, wup_ref, w1_ref, b1_ref, w2_ref, b2_ref, w3_ref, b3_ref,
              wout_ref, o_ref):
    h = jnp.dot(p_ref[...], wup_ref[...], preferred_element_type=jnp.float32)
    for w_r, b_r in ((w1_ref, b1_ref), (w2_ref, b2_ref), (w3_ref, b3_ref)):
        z = jnp.dot(h, w_r[...], preferred_element_type=jnp.float32) + b_r[...]
        h = z * jax.nn.sigmoid(z)
    o_ref[...] = jnp.dot(h, wout_ref[...], preferred_element_type=jnp.float32)


def _sc_segment_sum(h, idx01, b0v, b1v, zeros):
    e = h.shape[0]
    n_chunks = e // CHUNK
    mesh = plsc.VectorSubcoreMesh(core_axis_name="c", subcore_axis_name="s")

    @functools.partial(
        pl.kernel,
        mesh=mesh,
        compiler_params=pltpu.CompilerParams(use_tc_tiling_on_sc=True,
                                             needs_layout_passes=False),
        out_type=jax.ShapeDtypeStruct((N_PAD, FEAT), jnp.float32),
        scratch_types=[
            pltpu.VMEM((2, CHUNK, FEAT), jnp.float32),
            pltpu.VMEM((2 * MAX_TILE_CHUNKS, 128), jnp.int32),
            pltpu.VMEM((16,), jnp.int32),
            pltpu.VMEM_SHARED((ACC_ROWS, FEAT), jnp.float32),
            pltpu.SemaphoreType.DMA,
            pltpu.SemaphoreType.DMA,
            pltpu.SemaphoreType.DMA,
            pltpu.SemaphoreType.DMA,
            pltpu.SemaphoreType.DMA,
        ],
    )
    def run(h_hbm, idx_hbm, b0_hbm, b1_hbm, z_hbm, out_hbm, hbuf, idxbuf,
            bvec, acc, sem0, sem1, ssem0, ssem1, isem):
        c = lax.axis_index("c")
        s = lax.axis_index("s")
        r0 = s * ZSUB
        sems = (sem0, sem1)
        ssems = (ssem0, ssem1)

        pltpu.sync_copy(b0_hbm, bvec)
        nb0 = jnp.sum(bvec[...])
        pltpu.sync_copy(b1_hbm, bvec)
        nb1 = jnp.sum(bvec[...])

        start = jnp.where(c == 0, 0, nb1)
        end = jnp.where(c == 0, nb0, n_chunks)
        length = jnp.maximum(end - start, 0)
        per = ((length + 15) // 16 + 3) // 4 * 4
        t0 = start + s * per
        t_iters = jnp.minimum(per, jnp.maximum(end - t0, 0))

        def start_load(b, j):
            pltpu.async_copy(h_hbm.at[pl.ds((t0 + j) * CHUNK, CHUNK)],
                             hbuf.at[b], sems[b])

        def wait_load(b):
            pltpu.make_async_copy(h_hbm.at[pl.ds(0, CHUNK)], hbuf.at[b],
                                  sems[b]).wait()

        def wait_scat(b, j):
            for k in range(KSUB):
                pltpu.make_async_copy(hbuf.at[b, pl.ds(k * 128, 128)],
                                      acc.at[idxbuf.at[KSUB * j + k]],
                                      ssems[b]).wait()

        def do_scat(b, j):
            for k in range(KSUB):
                pltpu.async_copy(hbuf.at[b, pl.ds(k * 128, 128)],
                                 acc.at[idxbuf.at[KSUB * j + k]], ssems[b],
                                 add=True)

        irow = pl.multiple_of(c * IDX_ROWS_CORE + KSUB * t0, 8)
        pltpu.async_copy(
            idx_hbm.at[pl.ds(irow, KSUB * MAX_TILE_CHUNKS)], idxbuf, isem)

        @pl.when(t_iters > 0)
        def _():
            start_load(0, 0)

        pltpu.sync_copy(z_hbm.at[pl.ds(r0, ZSUB)], acc.at[pl.ds(r0, ZSUB)])
        pltpu.make_async_copy(
            idx_hbm.at[pl.ds(0, KSUB * MAX_TILE_CHUNKS)], idxbuf, isem).wait()
        plsc.subcore_barrier()

        def body(jj, carry):
            for b in (0, 1):
                j = jj * 2 + b

                @pl.when(j < t_iters)
                def _():
                    wait_load(b)

                    @pl.when(j >= 1)
                    def _():
                        wait_scat(b ^ 1, j - 1)

                    @pl.when(j + 1 < t_iters)
                    def _():
                        start_load(b ^ 1, j + 1)

                    do_scat(b, j)

            return carry

        lax.fori_loop(0, (per + 1) // 2, body, 0)
        last_j = t_iters - 1
        last = last_j % 2

        @pl.when(t_iters > 0)
        def _():
            @pl.when(last == 0)
            def _():
                wait_scat(0, last_j)

            @pl.when(last == 1)
            def _():
                wait_scat(1, last_j)

        plsc.subcore_barrier()
        pltpu.sync_copy(acc.at[pl.ds(r0, ZSUB)],
                        out_hbm.at[pl.ds(c * HALF + r0, ZSUB)])

    return run(h, idx01, b0v, b1v, zeros)


def kernel(x, rbf, i, num_nodes, W_rbf, W_up, W1, b1, W2, b2, W3, b3, W_out):
    e = x.shape[0]
    nr = rbf.shape[1]

    rbf_t = jnp.pad(rbf.T, ((0, 8 - nr), (0, 0)))
    wrbf_t = jnp.pad(W_rbf.T, ((0, 8 - nr), (0, 0)))

    h = pl.pallas_call(
        _edge_body,
        grid=(e // EDGE_BLOCK,),
        in_specs=[
            pl.BlockSpec((8, EDGE_BLOCK), lambda g: (0, g)),
            pl.BlockSpec((EDGE_BLOCK, FEAT), lambda g: (g, 0)),
            pl.BlockSpec((8, FEAT), lambda g: (0, 0)),
        ],
        out_specs=pl.BlockSpec((EDGE_BLOCK, FEAT), lambda g: (g, 0)),
        out_shape=jax.ShapeDtypeStruct((e, FEAT), jnp.float32),
    )(rbf_t, x, wrbf_t)

    idx = i.astype(jnp.int32)
    idx0 = jnp.where(idx < HALF, idx, HALF)
    idx1 = jnp.where(idx >= HALF, idx - HALF, HALF)
    pad_rows = jnp.zeros((IDX_ROWS_CORE - idx.shape[0] // 128, 128), jnp.int32)
    idx01 = jnp.concatenate([idx0.reshape(-1, 128), pad_rows,
                             idx1.reshape(-1, 128), pad_rows])
    b_edge = jnp.sum((idx < HALF).astype(jnp.int32))
    nb0 = (b_edge + CHUNK - 1) // CHUNK
    nb1 = b_edge // CHUNK // 4 * 4
    lane0 = (jax.lax.iota(jnp.int32, 16) == 0).astype(jnp.int32)
    b0v = lane0 * nb0
    b1v = lane0 * nb1

    zeros = jnp.zeros((HALF, FEAT), jnp.float32)
    partial = _sc_segment_sum(h, idx01, b0v, b1v, zeros)

    out = pl.pallas_call(
        _mlp_body,
        grid=(N_NODES // NODE_BLOCK,),
        in_specs=[
            pl.BlockSpec((NODE_BLOCK, FEAT), lambda g: (g, 0)),
            pl.BlockSpec((FEAT, OE), lambda g: (0, 0)),
            pl.BlockSpec((OE, OE), lambda g: (0, 0)),
            pl.BlockSpec((1, OE), lambda g: (0, 0)),
            pl.BlockSpec((OE, OE), lambda g: (0, 0)),
            pl.BlockSpec((1, OE), lambda g: (0, 0)),
            pl.BlockSpec((OE, OE), lambda g: (0, 0)),
            pl.BlockSpec((1, OE), lambda g: (0, 0)),
            pl.BlockSpec((OE, 1), lambda g: (0, 0)),
        ],
        out_specs=pl.BlockSpec((NODE_BLOCK, 1), lambda g: (g, 0)),
        out_shape=jax.ShapeDtypeStruct((N_NODES, 1), jnp.float32),
    )(partial, W_up.T, W1.T, b1.reshape(1, OE), W2.T, b2.reshape(1, OE),
      W3.T, b3.reshape(1, OE), W_out.T)
    return out


# speedup vs baseline: 1.0267x; 1.0267x over previous
import functools

import jax
import jax.numpy as jnp
from jax import lax
from jax.experimental import pallas as pl
from jax.experimental.pallas import tpu as pltpu
from jax.experimental.pallas import tpu_sc as plsc

N_NODES = 10000
FEAT = 128
OE = 256
EDGE_BLOCK = 12800
NODE_BLOCK = 2000
CHUNK = 256
KSUB = CHUNK // 128
N_TILES = 32
N_PAD = 10240
HALF = N_PAD // 2
ACC_ROWS = HALF + 8
ZSUB = HALF // 16


def _edge_body(rbft_ref, x_ref, w_ref, h_ref):
    s = lax.dot_general(rbft_ref[...], w_ref[...],
                        dimension_numbers=(((0,), (0,)), ((), ())),
                        preferred_element_type=jnp.float32)
    h_ref[...] = s * x_ref[...]


def _mlp_body(p_ref, wup_ref, w1_ref, b1_ref, w2_ref, b2_ref, w3_ref, b3_ref,
              wout_ref, o_ref):
    h = jnp.dot(p_ref[...], wup_ref[...], preferred_element_type=jnp.float32)
    for w_r, b_r in ((w1_ref, b1_ref), (w2_ref, b2_ref), (w3_ref, b3_ref)):
        z = jnp.dot(h, w_r[...], preferred_element_type=jnp.float32) + b_r[...]
        h = z * jax.nn.sigmoid(z)
    o_ref[...] = jnp.dot(h, wout_ref[...], preferred_element_type=jnp.float32)


def _sc_segment_sum(h, idx01, b0v, b1v, zeros):
    e = h.shape[0]
    n_chunks = e // CHUNK
    mesh = plsc.VectorSubcoreMesh(core_axis_name="c", subcore_axis_name="s")

    @functools.partial(
        pl.kernel,
        mesh=mesh,
        compiler_params=pltpu.CompilerParams(use_tc_tiling_on_sc=True,
                                             needs_layout_passes=False),
        out_type=jax.ShapeDtypeStruct((N_PAD, FEAT), jnp.float32),
        scratch_types=[
            pltpu.VMEM((2, CHUNK, FEAT), jnp.float32),
            pltpu.VMEM((2, KSUB, 128), jnp.int32),
            pltpu.VMEM((16,), jnp.int32),
            pltpu.VMEM_SHARED((ACC_ROWS, FEAT), jnp.float32),
            pltpu.SemaphoreType.DMA,
            pltpu.SemaphoreType.DMA,
            pltpu.SemaphoreType.DMA,
            pltpu.SemaphoreType.DMA,
        ],
    )
    def run(h_hbm, idx_hbm, b0_hbm, b1_hbm, z_hbm, out_hbm, hbuf, idxbuf,
            bvec, acc, sem0, sem1, ssem0, ssem1):
        c = lax.axis_index("c")
        s = lax.axis_index("s")
        r0 = s * ZSUB
        sems = (sem0, sem1)
        ssems = (ssem0, ssem1)

        pltpu.sync_copy(b0_hbm, bvec)
        nb0 = jnp.sum(bvec[...])
        pltpu.sync_copy(b1_hbm, bvec)
        nb1 = jnp.sum(bvec[...])

        start = jnp.where(c == 0, 0, nb1)
        end = jnp.where(c == 0, nb0, n_chunks)
        length = jnp.maximum(end - start, 0)
        per = (length + 15) // 16
        t0 = start + s * per
        t_iters = jnp.minimum(per, jnp.maximum(end - t0, 0))

        def start_load(b, j):
            ch = t0 + j
            pltpu.async_copy(h_hbm.at[pl.ds(ch * CHUNK, CHUNK)], hbuf.at[b],
                             sems[b])
            for k in range(KSUB):
                pltpu.async_copy(
                    idx_hbm.at[pl.ds(c * e + ch * CHUNK + k * 128, 128)],
                    idxbuf.at[b, k], sems[b])

        def wait_load(b):
            pltpu.make_async_copy(h_hbm.at[pl.ds(0, CHUNK)], hbuf.at[b],
                                  sems[b]).wait()
            for k in range(KSUB):
                pltpu.make_async_copy(idx_hbm.at[pl.ds(0, 128)],
                                      idxbuf.at[b, k], sems[b]).wait()

        def wait_scat(b, j):
            for k in range(KSUB):
                pltpu.make_async_copy(hbuf.at[b, pl.ds(k * 128, 128)],
                                      acc.at[idxbuf.at[b, k]], ssems[b]).wait()

        def do_scat(b, j):
            for k in range(KSUB):
                pltpu.async_copy(hbuf.at[b, pl.ds(k * 128, 128)],
                                 acc.at[idxbuf.at[b, k]], ssems[b], add=True)

        @pl.when(t_iters > 0)
        def _():
            start_load(0, 0)

        pltpu.sync_copy(z_hbm.at[pl.ds(r0, ZSUB)], acc.at[pl.ds(r0, ZSUB)])
        plsc.subcore_barrier()

        def body(jj, carry):
            for b in (0, 1):
                j = jj * 2 + b

                @pl.when(j < t_iters)
                def _():
                    wait_load(b)

                    @pl.when(j >= 1)
                    def _():
                        wait_scat(b ^ 1, j - 1)

                    @pl.when(j + 1 < t_iters)
                    def _():
                        start_load(b ^ 1, j + 1)

                    do_scat(b, j)

            return carry

        lax.fori_loop(0, (per + 1) // 2, body, 0)
        last_j = t_iters - 1
        last = last_j % 2

        @pl.when(t_iters > 0)
        def _():
            @pl.when(last == 0)
            def _():
                wait_scat(0, last_j)

            @pl.when(last == 1)
            def _():
                wait_scat(1, last_j)

        plsc.subcore_barrier()
        pltpu.sync_copy(acc.at[pl.ds(r0, ZSUB)],
                        out_hbm.at[pl.ds(c * HALF + r0, ZSUB)])

    return run(h, idx01, b0v, b1v, zeros)


def kernel(x, rbf, i, num_nodes, W_rbf, W_up, W1, b1, W2, b2, W3, b3, W_out):
    e = x.shape[0]
    nr = rbf.shape[1]

    rbf_t = jnp.pad(rbf.T, ((0, 8 - nr), (0, 0)))
    wrbf_t = jnp.pad(W_rbf.T, ((0, 8 - nr), (0, 0)))

    h = pl.pallas_call(
        _edge_body,
        grid=(e // EDGE_BLOCK,),
        in_specs=[
            pl.BlockSpec((8, EDGE_BLOCK), lambda g: (0, g)),
            pl.BlockSpec((EDGE_BLOCK, FEAT), lambda g: (g, 0)),
            pl.BlockSpec((8, FEAT), lambda g: (0, 0)),
        ],
        out_specs=pl.BlockSpec((EDGE_BLOCK, FEAT), lambda g: (g, 0)),
        out_shape=jax.ShapeDtypeStruct((e, FEAT), jnp.float32),
    )(rbf_t, x, wrbf_t)

    idx = i.astype(jnp.int32)
    idx0 = jnp.where(idx < HALF, idx, HALF)
    idx1 = jnp.where(idx >= HALF, idx - HALF, HALF)
    idx01 = jnp.concatenate([idx0, idx1])
    b_edge = jnp.sum((idx < HALF).astype(jnp.int32))
    nb0 = (b_edge + CHUNK - 1) // CHUNK
    nb1 = b_edge // CHUNK
    lane0 = (jax.lax.iota(jnp.int32, 16) == 0).astype(jnp.int32)
    b0v = lane0 * nb0
    b1v = lane0 * nb1

    zeros = jnp.zeros((HALF, FEAT), jnp.float32)
    partial = _sc_segment_sum(h, idx01, b0v, b1v, zeros)

    out = pl.pallas_call(
        _mlp_body,
        grid=(N_NODES // NODE_BLOCK,),
        in_specs=[
            pl.BlockSpec((NODE_BLOCK, FEAT), lambda g: (g, 0)),
            pl.BlockSpec((FEAT, OE), lambda g: (0, 0)),
            pl.BlockSpec((OE, OE), lambda g: (0, 0)),
            pl.BlockSpec((1, OE), lambda g: (0, 0)),
            pl.BlockSpec((OE, OE), lambda g: (0, 0)),
            pl.BlockSpec((1, OE), lambda g: (0, 0)),
            pl.BlockSpec((OE, OE), lambda g: (0, 0)),
            pl.BlockSpec((1, OE), lambda g: (0, 0)),
            pl.BlockSpec((OE, 1), lambda g: (0, 0)),
        ],
        out_specs=pl.BlockSpec((NODE_BLOCK, 1), lambda g: (g, 0)),
        out_shape=jax.ShapeDtypeStruct((N_NODES, 1), jnp.float32),
    )(partial, W_up.T, W1.T, b1.reshape(1, OE), W2.T, b2.reshape(1, OE),
      W3.T, b3.reshape(1, OE), W_out.T)
    return out
